# Initial kernel scaffold; baseline (speedup 1.0000x reference)
import jax
import jax.numpy as jnp
from jax.experimental import pallas as pl


def kernel(z_e, embedding):
    raise NotImplementedError("write your pallas kernel here")


# speedup vs baseline: 1.2892x; 1.2892x over previous
import functools

import jax
import jax.numpy as jnp
from jax import lax
from jax.experimental import pallas as pl
from jax.experimental.pallas import tpu as pltpu
from jax.experimental.pallas import tpu_sc as plsc

_NE = 1024
_D = 64
_BETA = 0.25
_N = 128 * 576

_TOK_BLOCK = 1024
_G = _N // _TOK_BLOCK

_NC = 2
_NS = 16
_NW = _NC * _NS
_ROWS_PER_W = _N // _NW
_CHUNK = 128
_NCHUNK = _ROWS_PER_W // _CHUNK


def _argmin_body(x_ref, e_ref, idx_ref, loss_ref):
    x = x_ref[...]
    e = e_ref[...]
    e2 = jnp.sum(e * e, axis=1)
    x2 = jnp.sum(x * x, axis=1, keepdims=True)
    prod = lax.dot_general(x, e, (((1,), (1,)), ((), ())),
                           preferred_element_type=jnp.float32)
    dist = x2 - 2.0 * prod + e2[None, :]
    minval = jnp.min(dist, axis=1, keepdims=True)
    ids = lax.broadcasted_iota(jnp.int32, dist.shape, 1)
    idx = jnp.min(jnp.where(dist == minval, ids, _NE), axis=1)
    idx_ref[0, 0, :] = idx

    @pl.when(pl.program_id(0) == 0)
    def _():
        loss_ref[...] = jnp.zeros((1, 1), jnp.float32)

    loss_ref[...] += jnp.sum(minval).reshape(1, 1)


_argmin_call = pl.pallas_call(
    _argmin_body,
    grid=(_G,),
    in_specs=[
        pl.BlockSpec((_TOK_BLOCK, _D), lambda i: (i, 0)),
        pl.BlockSpec((_NE, _D), lambda i: (0, 0)),
    ],
    out_specs=[
        pl.BlockSpec((1, 1, _TOK_BLOCK), lambda i: (i, 0, 0)),
        pl.BlockSpec((1, 1), lambda i: (0, 0)),
    ],
    out_shape=[
        jax.ShapeDtypeStruct((_G, 1, _TOK_BLOCK), jnp.int32),
        jax.ShapeDtypeStruct((1, 1), jnp.float32),
    ],
)


@functools.cache
def _make_gather_sc():
    def body(emb_hbm, idx3_hbm, out_hbm, idx_v, rows_v, gsem):
        wid = lax.axis_index("s") * _NC + lax.axis_index("c")
        base = wid * _ROWS_PER_W
        pltpu.sync_copy(idx3_hbm.at[wid], idx_v)
        for c in range(_NCHUNK):
            pltpu.async_copy(emb_hbm.at[idx_v.at[c]], rows_v, gsem).wait()
            pltpu.sync_copy(rows_v,
                            out_hbm.at[pl.ds(base + c * _CHUNK, _CHUNK)])

    return pl.kernel(
        body,
        out_type=jax.ShapeDtypeStruct((_N, _D), jnp.float32),
        mesh=plsc.VectorSubcoreMesh(core_axis_name="c", subcore_axis_name="s"),
        compiler_params=pltpu.CompilerParams(use_tc_tiling_on_sc=False),
        scratch_types=[
            pltpu.VMEM((_NCHUNK, _CHUNK), jnp.int32),
            pltpu.VMEM((_CHUNK, _D), jnp.float32),
            pltpu.SemaphoreType.DMA,
        ],
    )


def kernel(z_e, embedding):
    flat = z_e.reshape(_N, _D)
    idx3, loss_acc = _argmin_call(flat, embedding)
    idx_w = idx3.reshape(_NW, _NCHUNK, _CHUNK)
    z_q = _make_gather_sc()(embedding, idx_w)
    loss = loss_acc[0, 0] * ((1.0 + _BETA) / (_N * _D))
    return z_q.reshape(z_e.shape), loss


# speedup vs baseline: 1.7872x; 1.3863x over previous
import functools

import jax
import jax.numpy as jnp
from jax import lax
from jax.experimental import pallas as pl
from jax.experimental.pallas import tpu as pltpu
from jax.experimental.pallas import tpu_sc as plsc

_NE = 1024
_D = 64
_BETA = 0.25
_N = 128 * 576

_TOK_BLOCK = 1024
_G = _N // _TOK_BLOCK

_NC = 2
_NS = 16
_NW = _NC * _NS
_ROWS_PER_W = _N // _NW
_CHUNK = 128
_NCHUNK = _ROWS_PER_W // _CHUNK


def _argmin_body(x_ref, e_ref, idx_ref, loss_ref):
    x = x_ref[...]
    e = e_ref[...]
    e2 = jnp.sum(e * e, axis=1, keepdims=True)
    x2 = jnp.sum(x * x, axis=1)
    prod = lax.dot_general(e, x, (((1,), (1,)), ((), ())),
                           preferred_element_type=jnp.float32)
    dist = e2 - 2.0 * prod
    minval = jnp.min(dist, axis=0, keepdims=True)
    ids = lax.broadcasted_iota(jnp.int32, dist.shape, 0)
    idx = jnp.min(jnp.where(dist == minval, ids, _NE), axis=0)
    idx_ref[0, 0, :] = idx

    @pl.when(pl.program_id(0) == 0)
    def _():
        loss_ref[...] = jnp.zeros((1, 1), jnp.float32)

    loss_ref[...] += (jnp.sum(minval) + jnp.sum(x2)).reshape(1, 1)


_argmin_call = pl.pallas_call(
    _argmin_body,
    grid=(_G,),
    in_specs=[
        pl.BlockSpec((_TOK_BLOCK, _D), lambda i: (i, 0)),
        pl.BlockSpec((_NE, _D), lambda i: (0, 0)),
    ],
    out_specs=[
        pl.BlockSpec((1, 1, _TOK_BLOCK), lambda i: (i, 0, 0)),
        pl.BlockSpec((1, 1), lambda i: (0, 0)),
    ],
    out_shape=[
        jax.ShapeDtypeStruct((_G, 1, _TOK_BLOCK), jnp.int32),
        jax.ShapeDtypeStruct((1, 1), jnp.float32),
    ],
)


@functools.cache
def _make_gather_sc():
    def body(emb_hbm, idx3_hbm, out_hbm, idx_v, rows_v, gsem):
        wid = lax.axis_index("s") * _NC + lax.axis_index("c")
        base = wid * _ROWS_PER_W
        pltpu.sync_copy(idx3_hbm.at[wid], idx_v)
        for c in range(_NCHUNK):
            pltpu.async_copy(emb_hbm.at[idx_v.at[c]], rows_v, gsem).wait()
            pltpu.sync_copy(rows_v,
                            out_hbm.at[pl.ds(base + c * _CHUNK, _CHUNK)])

    return pl.kernel(
        body,
        out_type=jax.ShapeDtypeStruct((_N, _D), jnp.float32),
        mesh=plsc.VectorSubcoreMesh(core_axis_name="c", subcore_axis_name="s"),
        compiler_params=pltpu.CompilerParams(use_tc_tiling_on_sc=False),
        scratch_types=[
            pltpu.VMEM((_NCHUNK, _CHUNK), jnp.int32),
            pltpu.VMEM((_CHUNK, _D), jnp.float32),
            pltpu.SemaphoreType.DMA,
        ],
    )


def kernel(z_e, embedding):
    flat = z_e.reshape(_N, _D)
    idx3, loss_acc = _argmin_call(flat, embedding)
    idx_w = idx3.reshape(_NW, _NCHUNK, _CHUNK)
    z_q = _make_gather_sc()(embedding, idx_w)
    loss = loss_acc[0, 0] * ((1.0 + _BETA) / (_N * _D))
    return z_q.reshape(z_e.shape), loss
